# baseline (speedup 1.0000x reference)
import jax
import jax.numpy as jnp
from jax import lax
from jax.experimental import pallas as pl
from jax.experimental.pallas import tpu as pltpu
from jax.experimental.pallas import tpu_sc as plsc

BATCH = 16384
EMBED_DIM = 64
NUM_CORES = 2
NUM_SUBCORES = 16
NUM_WORKERS = NUM_CORES * NUM_SUBCORES
BPW = BATCH // NUM_WORKERS
CHUNK = 128
NCHUNK = BPW // CHUNK
LANES = 16
NGROUP = BPW // LANES


def _sc_body(users_hbm, items_hbm, uembt_hbm, iembt_hbm, ub_hbm, ib_hbm,
             gb_hbm, out_hbm,
             uidx_v, iidx_v, uval_v, ival_v, ubias_v, ibias_v, gb_v,
             out_v, sem, bsem):
    wid = lax.axis_index("s") * NUM_CORES + lax.axis_index("c")
    base = wid * BPW

    for j in range(NCHUNK):
        pltpu.sync_copy(users_hbm.at[pl.ds(base + j * CHUNK, CHUNK)],
                        uidx_v.at[j])
        pltpu.sync_copy(items_hbm.at[pl.ds(base + j * CHUNK, CHUNK)],
                        iidx_v.at[j])
    pltpu.sync_copy(gb_hbm, gb_v)

    for j in range(NCHUNK):
        sl = pl.ds(j * CHUNK, CHUNK)
        pltpu.async_copy(ub_hbm.at[uidx_v.at[j]], ubias_v.at[sl], bsem)
        pltpu.async_copy(ib_hbm.at[iidx_v.at[j]], ibias_v.at[sl], bsem)

    def fire(d):
        cs = []
        for j in range(NCHUNK):
            sl = pl.ds(j * CHUNK, CHUNK)
            cs.append(pltpu.async_copy(uembt_hbm.at[d].at[uidx_v.at[j]],
                                       uval_v.at[d, sl], sem))
            cs.append(pltpu.async_copy(iembt_hbm.at[d].at[iidx_v.at[j]],
                                       ival_v.at[d, sl], sem))
        return cs

    def d_body(d, _):
        for c in fire(d):
            c.wait()
        return 0

    lax.fori_loop(0, EMBED_DIM, d_body, 0)

    iota = jnp.arange(LANES, dtype=jnp.int32)
    gbias = gb_v[...]

    def group_body(g, _):
        sl = pl.ds(g * LANES, LANES)
        acc = jnp.zeros((LANES,), dtype=jnp.float32)
        for d in range(EMBED_DIM):
            acc = acc + uval_v[d, sl] * ival_v[d, sl]
        return_val = acc + ubias_v[sl] + ibias_v[sl] + gbias
        out_v[sl] = return_val
        return 0

    for j in range(NCHUNK):
        sl = pl.ds(j * CHUNK, CHUNK)
        pltpu.make_async_copy(ub_hbm.at[uidx_v.at[j]], ubias_v.at[sl],
                              bsem).wait()
        pltpu.make_async_copy(ib_hbm.at[iidx_v.at[j]], ibias_v.at[sl],
                              bsem).wait()

    lax.fori_loop(0, NGROUP, group_body, 0)

    pltpu.sync_copy(out_v, out_hbm.at[pl.ds(base, BPW)])


@jax.jit
def _bpr_scores(users, items, uembt, iembt, user_bias, item_bias, gb16):
    mesh = plsc.VectorSubcoreMesh(core_axis_name="c", subcore_axis_name="s",
                                  num_cores=NUM_CORES,
                                  num_subcores=NUM_SUBCORES)
    f = pl.kernel(
        _sc_body,
        out_type=jax.ShapeDtypeStruct((BATCH,), jnp.float32),
        mesh=mesh,
        compiler_params=pltpu.CompilerParams(needs_layout_passes=False,
                                             use_tc_tiling_on_sc=False),
        scratch_types=[
            pltpu.VMEM((NCHUNK, CHUNK), jnp.int32),
            pltpu.VMEM((NCHUNK, CHUNK), jnp.int32),
            pltpu.VMEM((EMBED_DIM, BPW), jnp.float32),
            pltpu.VMEM((EMBED_DIM, BPW), jnp.float32),
            pltpu.VMEM((BPW,), jnp.float32),
            pltpu.VMEM((BPW,), jnp.float32),
            pltpu.VMEM((LANES,), jnp.float32),
            pltpu.VMEM((BPW,), jnp.float32),
            pltpu.SemaphoreType.DMA,
            pltpu.SemaphoreType.DMA,
        ],
    )
    return f(users, items, uembt, iembt, user_bias, item_bias, gb16)


def kernel(users, items, user_emb_w, item_emb_w, user_bias_w, item_bias_w,
           global_bias):
    users = users.astype(jnp.int32)
    items = items.astype(jnp.int32)
    gb16 = jnp.broadcast_to(global_bias.reshape(()), (16,))
    return _bpr_scores(users, items, user_emb_w.T, item_emb_w.T,
                       user_bias_w.reshape(-1), item_bias_w.reshape(-1),
                       gb16)
